# Initial kernel scaffold; baseline (speedup 1.0000x reference)
import jax
import jax.numpy as jnp
from jax.experimental import pallas as pl


def kernel(x, adj, mask, adj_o, features_o, mask_internal_edges, Wg1, bg1, Wg2, bg2, W11, b11, W12, b12, W13, b13, W14, b14, W15, b15, W16, b16, W21, b21, W22, b22, W23, b23):
    raise NotImplementedError("write your pallas kernel here")


# speedup vs baseline: 13.8686x; 13.8686x over previous
import functools

import jax
import jax.numpy as jnp
from jax import lax
from jax.experimental import pallas as pl
from jax.experimental.pallas import tpu as pltpu

OLD = 512
N_TOT = 1024
N_NEW = 512
BUDGET = 2000
FEAT_BUDGET = 500
NEG = -1e20


def _mm(a, b):
    return jnp.dot(a, b, preferred_element_type=jnp.float32)


def _kth_threshold_bits(bits, k):
    def body(_, lohi):
        lo, hi = lohi
        mid = (lo + hi) // 2
        cnt = jnp.sum((bits >= mid).astype(jnp.int32))
        take = cnt >= k
        return (jnp.where(take, mid, lo), jnp.where(take, hi, mid))

    lo = jnp.int32(0)
    hi = jnp.int32(0x40000000)
    lo, hi = lax.fori_loop(0, 31, body, (lo, hi))
    return lo


def _encoder_kernel(x_ref, adj_ref, wg1_ref, bg1_ref, wg2_ref, bg2_ref,
                    w21_ref, b21_ref, w22_ref, b22_ref, w23_ref, b23_ref,
                    rx_ref):
    xw = _mm(x_ref[...], wg1_ref[...])
    h = jax.nn.relu(_mm(adj_ref[...], xw) + bg1_ref[...])
    hw = _mm(h, wg2_ref[...])
    z_new = jax.nn.relu(_mm(adj_ref[OLD:, :], hw) + bg2_ref[...])
    r = jax.nn.relu(_mm(z_new, w21_ref[...]) + b21_ref[...])
    r = jax.nn.relu(_mm(r, w22_ref[...]) + b22_ref[...])
    s = _mm(r, w23_ref[...]) + b23_ref[...]
    e = jnp.exp(s - jnp.max(s))
    bits = lax.bitcast_convert_type(e, jnp.int32)
    thr = _kth_threshold_bits(bits, FEAT_BUDGET)
    rx_ref[...] = (bits >= thr).astype(jnp.float32)


def _edge_mlp_kernel(rx_ref, v_ref, valid_ref,
                     w11_ref, b11_ref, w12_ref, b12_ref, w13_ref, b13_ref,
                     w14_ref, b14_ref, w15_ref, b15_ref, w16_ref, b16_ref,
                     s_ref, *, bi):
    v = v_ref[...]
    w16r = w16_ref[...].reshape(1, -1)
    b16 = b16_ref[0, 0]
    f = (rx_ref[...][:, None, :] * v[None, :, :]).reshape(bi * N_TOT, 32)
    a = jax.nn.relu(_mm(f, w11_ref[...]) + b11_ref[...])
    a = jax.nn.relu(_mm(a, w12_ref[...]) + b12_ref[...])
    a = jax.nn.relu(_mm(a, w13_ref[...]) + b13_ref[...])
    a = jax.nn.relu(_mm(a, w14_ref[...]) + b14_ref[...])
    a = jax.nn.relu(_mm(a, w15_ref[...]) + b15_ref[...])
    score = (_mm(a, w16_ref[...]) + b16).reshape(bi, N_TOT)
    s_ref[...] = jnp.where(valid_ref[...] > 0, score, NEG)


def _finalize_kernel(s_ref, internal_ref, adj_o_ref, out_ref):
    s = s_ref[...]
    e = jnp.exp(s - jnp.max(s))
    bits = lax.bitcast_convert_type(e, jnp.int32)
    thr = _kth_threshold_bits(bits, BUDGET)
    hard = (bits >= thr).astype(jnp.float32)
    m = jnp.where(internal_ref[...] > 0, 1.0, hard)

    r5 = lax.broadcasted_iota(jnp.int32, (OLD, OLD), 0)
    c5 = lax.broadcasted_iota(jnp.int32, (OLD, OLD), 1)
    eye = (r5 == c5).astype(jnp.float32)
    strict_low = (r5 > c5).astype(jnp.float32)

    ao = adj_o_ref[...]
    t = ao * strict_low
    out_ref[:OLD, :OLD] = t + t.T + eye

    ll = m[:, :OLD]
    out_ref[OLD:, :OLD] = ll
    out_ref[:OLD, OLD:] = ll.T

    m2 = m[:, OLD:]
    low2 = m2 * strict_low
    out_ref[OLD:, OLD:] = low2 + low2.T + eye


def kernel(x, adj, mask, adj_o, features_o, mask_internal_edges,
           Wg1, bg1, Wg2, bg2,
           W11, b11, W12, b12, W13, b13, W14, b14, W15, b15, W16, b16,
           W21, b21, W22, b22, W23, b23):
    f32 = jnp.float32
    bg1r = bg1.reshape(1, -1)
    bg2r = bg2.reshape(1, -1)
    b21r = b21.reshape(1, -1)
    b22r = b22.reshape(1, -1)
    b23r = b23.reshape(1, -1)
    b11r = b11.reshape(1, -1)
    b12r = b12.reshape(1, -1)
    b13r = b13.reshape(1, -1)
    b14r = b14.reshape(1, -1)
    b15r = b15.reshape(1, -1)
    b16r = b16.reshape(1, 1)

    rx = pl.pallas_call(
        _encoder_kernel,
        out_shape=jax.ShapeDtypeStruct((N_NEW, 32), f32),
    )(x, adj, Wg1, bg1r, Wg2, bg2r, W21, b21r, W22, b22r, W23, b23r)

    recon_X = jnp.concatenate([features_o, rx], axis=0)

    valid = (mask & jnp.logical_not(mask_internal_edges)).astype(f32)
    internal = mask_internal_edges.astype(f32)

    BI = 8
    grid = N_NEW // BI

    def _full(a):
        return pl.BlockSpec(a.shape, lambda i: (0,) * a.ndim)

    scores = pl.pallas_call(
        functools.partial(_edge_mlp_kernel, bi=BI),
        grid=(grid,),
        in_specs=[
            pl.BlockSpec((BI, 32), lambda i: (i, 0)),
            pl.BlockSpec((N_TOT, 32), lambda i: (0, 0)),
            pl.BlockSpec((BI, N_TOT), lambda i: (i, 0)),
            _full(W11), _full(b11r), _full(W12), _full(b12r),
            _full(W13), _full(b13r), _full(W14), _full(b14r),
            _full(W15), _full(b15r), _full(W16), _full(b16r),
        ],
        out_specs=pl.BlockSpec((BI, N_TOT), lambda i: (i, 0)),
        out_shape=jax.ShapeDtypeStruct((N_NEW, N_TOT), f32),
    )(rx, recon_X, valid,
      W11, b11r, W12, b12r, W13, b13r, W14, b14r, W15, b15r, W16, b16r)

    ra = pl.pallas_call(
        _finalize_kernel,
        out_shape=jax.ShapeDtypeStruct((N_TOT, N_TOT), f32),
    )(scores, internal, adj_o)

    return ra, recon_X


# speedup vs baseline: 26.1232x; 1.8836x over previous
import functools

import jax
import jax.numpy as jnp
from jax import lax
from jax.experimental import pallas as pl
from jax.experimental.pallas import tpu as pltpu

OLD = 512
N_TOT = 1024
N_NEW = 512
BUDGET = 2000
FEAT_BUDGET = 500
NEG = -1e20


def _mm(a, b):
    return jnp.dot(a, b, preferred_element_type=jnp.float32)


def _kth_threshold_bits(bits, k):
    def body(_, lohi):
        lo, hi = lohi
        mid = (lo + hi) // 2
        cnt = jnp.sum((bits >= mid).astype(jnp.int32))
        take = cnt >= k
        return (jnp.where(take, mid, lo), jnp.where(take, hi, mid))

    lo = jnp.int32(0)
    hi = jnp.int32(0x40000000)
    lo, hi = lax.fori_loop(0, 31, body, (lo, hi))
    return lo


def _encoder_kernel(x_ref, adj_ref, wg1_ref, bg1_ref, wg2_ref, bg2_ref,
                    w21_ref, b21_ref, w22_ref, b22_ref, w23_ref, b23_ref,
                    rx_ref):
    xw = _mm(x_ref[...], wg1_ref[...])
    h = jax.nn.relu(_mm(adj_ref[...], xw) + bg1_ref[...])
    hw = _mm(h, wg2_ref[...])
    z_new = jax.nn.relu(_mm(adj_ref[OLD:, :], hw) + bg2_ref[...])
    r = jax.nn.relu(_mm(z_new, w21_ref[...]) + b21_ref[...])
    r = jax.nn.relu(_mm(r, w22_ref[...]) + b22_ref[...])
    s = _mm(r, w23_ref[...]) + b23_ref[...]
    e = jnp.exp(s - jnp.max(s))
    bits = lax.bitcast_convert_type(e, jnp.int32)
    thr = _kth_threshold_bits(bits, FEAT_BUDGET)
    rx_ref[...] = (bits >= thr).astype(jnp.float32)


def _edge_mlp_kernel(rx_ref, v_ref,
                     w11_ref, b11_ref, w12_ref, b12_ref, w13_ref, b13_ref,
                     w14_ref, b14_ref, w15_ref, b15_ref, w16_ref, b16_ref,
                     s_ref, *, bi):
    v = v_ref[...]
    b16 = b16_ref[0, 0]
    pieces = []
    for g in range(bi // 4):
        f4 = jnp.concatenate(
            [v * rx_ref[4 * g + k:4 * g + k + 1, :] for k in range(4)],
            axis=1)
        a = jax.nn.relu(_mm(f4, w11_ref[...]) + b11_ref[...])
        a = jax.nn.relu(_mm(a, w12_ref[...]) + b12_ref[...])
        a = jax.nn.relu(_mm(a, w13_ref[...]) + b13_ref[...])
        a = jax.nn.relu(_mm(a, w14_ref[...]) + b14_ref[...])
        a = jax.nn.relu(_mm(a, w15_ref[...]) + b15_ref[...])
        pieces.append(_mm(a, w16_ref[...]) + b16)
    s_ref[...] = jnp.concatenate(pieces, axis=1)


def _finalize_kernel(s_ref, valid_ref, internal_ref, adj_o_ref, out_ref):
    s = jnp.where(valid_ref[...] > 0, s_ref[...], NEG)
    e = jnp.exp(s - jnp.max(s))
    bits = lax.bitcast_convert_type(e, jnp.int32)
    thr = _kth_threshold_bits(bits, BUDGET)
    hard = (bits >= thr).astype(jnp.float32)
    mt = jnp.where(internal_ref[...] > 0, 1.0, hard)

    r5 = lax.broadcasted_iota(jnp.int32, (OLD, OLD), 0)
    c5 = lax.broadcasted_iota(jnp.int32, (OLD, OLD), 1)
    eye = (r5 == c5).astype(jnp.float32)
    strict_low = (r5 > c5).astype(jnp.float32)
    strict_up = (r5 < c5).astype(jnp.float32)

    ao = adj_o_ref[...]
    t = ao * strict_low
    out_ref[:OLD, :OLD] = t + t.T + eye

    ur = mt[:OLD, :]
    out_ref[:OLD, OLD:] = ur
    out_ref[OLD:, :OLD] = ur.T

    u = mt[OLD:, :] * strict_up
    out_ref[OLD:, OLD:] = u + u.T + eye


def kernel(x, adj, mask, adj_o, features_o, mask_internal_edges,
           Wg1, bg1, Wg2, bg2,
           W11, b11, W12, b12, W13, b13, W14, b14, W15, b15, W16, b16,
           W21, b21, W22, b22, W23, b23):
    f32 = jnp.float32
    bg1r = bg1.reshape(1, -1)
    bg2r = bg2.reshape(1, -1)
    b21r = b21.reshape(1, -1)
    b22r = b22.reshape(1, -1)
    b23r = b23.reshape(1, -1)
    eye4 = jnp.eye(4, dtype=f32)
    W11k = jnp.kron(eye4, W11)
    W12k = jnp.kron(eye4, W12)
    W13k = jnp.kron(eye4, W13)
    W14k = jnp.kron(eye4, W14)
    W15k = jnp.kron(eye4, W15)
    W16k = jnp.kron(eye4, W16)
    b11r = jnp.tile(b11.reshape(1, -1), (1, 4))
    b12r = jnp.tile(b12.reshape(1, -1), (1, 4))
    b13r = jnp.tile(b13.reshape(1, -1), (1, 4))
    b14r = jnp.tile(b14.reshape(1, -1), (1, 4))
    b15r = jnp.tile(b15.reshape(1, -1), (1, 4))
    b16r = b16.reshape(1, 1)

    rx = pl.pallas_call(
        _encoder_kernel,
        out_shape=jax.ShapeDtypeStruct((N_NEW, 32), f32),
    )(x, adj, Wg1, bg1r, Wg2, bg2r, W21, b21r, W22, b22r, W23, b23r)

    recon_X = jnp.concatenate([features_o, rx], axis=0)

    valid_t = (mask & jnp.logical_not(mask_internal_edges)).T.astype(f32)
    internal_t = mask_internal_edges.T.astype(f32)

    BI = 128
    grid = N_NEW // BI

    def _full(a):
        return pl.BlockSpec(a.shape, lambda i: (0,) * a.ndim)

    scores_t = pl.pallas_call(
        functools.partial(_edge_mlp_kernel, bi=BI),
        grid=(grid,),
        in_specs=[
            pl.BlockSpec((BI, 32), lambda i: (i, 0)),
            pl.BlockSpec((N_TOT, 32), lambda i: (0, 0)),
            _full(W11k), _full(b11r), _full(W12k), _full(b12r),
            _full(W13k), _full(b13r), _full(W14k), _full(b14r),
            _full(W15k), _full(b15r), _full(W16k), _full(b16r),
        ],
        out_specs=pl.BlockSpec((N_TOT, BI), lambda i: (0, i)),
        out_shape=jax.ShapeDtypeStruct((N_TOT, N_NEW), f32),
    )(rx, recon_X,
      W11k, b11r, W12k, b12r, W13k, b13r, W14k, b14r, W15k, b15r,
      W16k, b16r)

    ra = pl.pallas_call(
        _finalize_kernel,
        out_shape=jax.ShapeDtypeStruct((N_TOT, N_TOT), f32),
    )(scores_t, valid_t, internal_t, adj_o)

    return ra, recon_X
